# baseline (speedup 1.0000x reference)
import functools

import jax
import jax.numpy as jnp
from jax import lax
from jax.experimental import pallas as pl
from jax.experimental.pallas import tpu as pltpu
from jax.experimental.pallas import tpu_sc as plsc


def _pq_body(x_ref, ws_ref, wt_ref, p_ref, q_ref):
    xv = x_ref[...]
    dn = (((1,), (1,)), ((), ()))
    p_ref[...] = lax.dot_general(xv, ws_ref[...], dn,
                                 preferred_element_type=jnp.float32)
    q_ref[...] = lax.dot_general(xv, wt_ref[...], dn,
                                 preferred_element_type=jnp.float32)


def _compute_pq(x, ws, wt):
    n, d = x.shape
    return pl.pallas_call(
        _pq_body,
        out_shape=(
            jax.ShapeDtypeStruct((n, d), jnp.float32),
            jax.ShapeDtypeStruct((n, d), jnp.float32),
        ),
    )(x, ws, wt)


def _make_gather_add(n_edges, d, chunk):
    info = plsc.get_sparse_core_info()
    nc, ns = info.num_cores, info.num_subcores
    nw = nc * ns
    per_w = n_edges // nw
    n_chunks = per_w // chunk
    n_pairs = (n_chunks + 2) // 2
    assert per_w % chunk == 0 and n_edges % nw == 0
    assert chunk % 8 == 0 and chunk <= 128

    mesh = plsc.VectorSubcoreMesh(core_axis_name="c", subcore_axis_name="s")

    @functools.partial(
        pl.kernel,
        mesh=mesh,
        out_type=jax.ShapeDtypeStruct((n_edges, d), jnp.float32),
        scratch_types=[
            pltpu.VMEM((chunk,), jnp.int32),
            pltpu.VMEM((chunk,), jnp.int32),
            pltpu.VMEM((chunk,), jnp.int32),
            pltpu.VMEM((chunk,), jnp.int32),
            pltpu.VMEM((chunk, d), jnp.float32),
            pltpu.VMEM((chunk, d), jnp.float32),
            pltpu.VMEM((chunk, d), jnp.float32),
            pltpu.VMEM((chunk, d), jnp.float32),
        ] + [pltpu.SemaphoreType.DMA] * 10,
    )
    def gather_add(p_hbm, q_hbm, src_hbm, tgt_hbm, out_hbm,
                   is0, is1, it0, it1, rp0, rp1, rq0, rq1,
                   sis0, sis1, sit0, sit1, sgp0, sgp1, sgq0, sgq1, sw0, sw1):
        idx_s = [is0, is1]
        idx_t = [it0, it1]
        rows_p = [rp0, rp1]
        rows_q = [rq0, rq1]
        sem_is = [sis0, sis1]
        sem_it = [sit0, sit1]
        sem_gp = [sgp0, sgp1]
        sem_gq = [sgq0, sgq1]
        sem_w = [sw0, sw1]

        wid = lax.axis_index("s") * nc + lax.axis_index("c")
        base = wid * per_w

        def fire_idx(c, b):
            eb = base + c * chunk
            pltpu.async_copy(src_hbm.at[pl.ds(eb, chunk)], idx_s[b], sem_is[b])
            pltpu.async_copy(tgt_hbm.at[pl.ds(eb, chunk)], idx_t[b], sem_it[b])

        def wait_idx(c, b):
            eb = base + c * chunk
            pltpu.make_async_copy(
                src_hbm.at[pl.ds(eb, chunk)], idx_s[b], sem_is[b]).wait()
            pltpu.make_async_copy(
                tgt_hbm.at[pl.ds(eb, chunk)], idx_t[b], sem_it[b]).wait()

        def fire_gather(b):
            pltpu.async_copy(p_hbm.at[idx_s[b]], rows_p[b], sem_gp[b])
            pltpu.async_copy(q_hbm.at[idx_t[b]], rows_q[b], sem_gq[b])

        def wait_gather(b):
            pltpu.make_async_copy(p_hbm.at[idx_s[b]], rows_p[b], sem_gp[b]).wait()
            pltpu.make_async_copy(q_hbm.at[idx_t[b]], rows_q[b], sem_gq[b]).wait()

        def fire_wb(c, b):
            eb = base + c * chunk
            pltpu.async_copy(rows_p[b], out_hbm.at[pl.ds(eb, chunk)], sem_w[b])

        def wait_wb(c, b):
            eb = base + c * chunk
            pltpu.make_async_copy(
                rows_p[b], out_hbm.at[pl.ds(eb, chunk)], sem_w[b]).wait()

        fire_idx(0, 0)
        fire_idx(1, 1)
        wait_idx(0, 0)
        fire_gather(0)

        def pair(hh, carry):
            for b in range(2):
                h = 2 * hh + b
                nb = 1 - b

                @pl.when(h < n_chunks)
                def _():
                    wait_gather(b)

                    @pl.when(h + 2 < n_chunks)
                    def _():
                        fire_idx(h + 2, b)

                    @pl.when(h >= 1)
                    def _():
                        wait_wb(h - 1, nb)

                    @pl.when(h + 1 < n_chunks)
                    def _():
                        wait_idx(h + 1, nb)
                        fire_gather(nb)

                    def add_row(i, c2):
                        for j in range(d // 16):
                            sl = pl.ds(j * 16, 16)
                            rows_p[b][i, sl] = rows_p[b][i, sl] + rows_q[b][i, sl]
                        return c2

                    lax.fori_loop(0, chunk, add_row, 0)
                    fire_wb(h, b)
            return carry

        lax.fori_loop(0, n_pairs, pair, 0)
        wait_wb(n_chunks - 1, (n_chunks - 1) % 2)

    return gather_add


def _msg_body(e_ref, g_ref, we_ref, o_ref):
    dn = (((1,), (1,)), ((), ()))
    o_ref[...] = g_ref[...] + lax.dot_general(
        e_ref[...], we_ref[...], dn, preferred_element_type=jnp.float32)


def _msg_body_aliased(e_ref, g_ref, we_ref, prev_ref, o_ref):
    del prev_ref
    _msg_body(e_ref, g_ref, we_ref, o_ref)


def _compute_msg_half(edge_full, g_half, we, block, block_off, prev=None):
    e_total, d = edge_full.shape
    grid = (g_half.shape[0] // block,)
    in_specs = [
        pl.BlockSpec((block, d), lambda i: (i + block_off, 0)),
        pl.BlockSpec((block, d), lambda i: (i, 0)),
        pl.BlockSpec((d, d), lambda i: (0, 0)),
    ]
    args = [edge_full, g_half, we]
    body = _msg_body
    kwargs = {}
    if prev is not None:
        in_specs.append(pl.BlockSpec(memory_space=pl.ANY))
        args.append(prev)
        body = _msg_body_aliased
        kwargs["input_output_aliases"] = {3: 0}
    return pl.pallas_call(
        body,
        grid=grid,
        in_specs=in_specs,
        out_specs=pl.BlockSpec((block, d), lambda i: (i + block_off, 0)),
        out_shape=jax.ShapeDtypeStruct((e_total, d), jnp.float32),
        **kwargs,
    )(*args)


def kernel(x, edge_index, edge_attr, W):
    n, d = x.shape
    e = edge_attr.shape[0]
    ws = W[:, :d]
    wt = W[:, d:2 * d]
    we = W[:, 2 * d:]

    p, q = _compute_pq(x, ws, wt)
    src = edge_index[0]
    tgt = edge_index[1]

    half = e // 2
    block = 3200
    gather_half = _make_gather_add(half, d, chunk=40)

    g_a = gather_half(p, q, src[:half], tgt[:half])
    g_b = gather_half(p, q, src[half:], tgt[half:])

    out_a = _compute_msg_half(edge_attr, g_a, we, block, 0)
    return _compute_msg_half(edge_attr, g_b, we, block,
                             half // block, prev=out_a)


# speedup vs baseline: 1.2080x; 1.2080x over previous
import functools

import jax
import jax.numpy as jnp
from jax import lax
from jax.experimental import pallas as pl
from jax.experimental.pallas import tpu as pltpu
from jax.experimental.pallas import tpu_sc as plsc


def _pq_body(x_ref, ws_ref, wt_ref, p_ref, q_ref):
    xv = x_ref[...]
    dn = (((1,), (1,)), ((), ()))
    p_ref[...] = lax.dot_general(xv, ws_ref[...], dn,
                                 preferred_element_type=jnp.float32)
    q_ref[...] = lax.dot_general(xv, wt_ref[...], dn,
                                 preferred_element_type=jnp.float32)


def _compute_pq(x, ws, wt):
    n, d = x.shape
    return pl.pallas_call(
        _pq_body,
        out_shape=(
            jax.ShapeDtypeStruct((n, d), jnp.float32),
            jax.ShapeDtypeStruct((n, d), jnp.float32),
        ),
    )(x, ws, wt)


def _make_gather_add(n_edges, d, chunk):
    info = plsc.get_sparse_core_info()
    nc, ns = info.num_cores, info.num_subcores
    nw = nc * ns
    per_w = n_edges // nw
    n_chunks = per_w // chunk
    n_pairs = (n_chunks + 2) // 2
    assert per_w % chunk == 0 and n_edges % nw == 0
    assert chunk % 8 == 0 and chunk <= 128

    mesh = plsc.VectorSubcoreMesh(core_axis_name="c", subcore_axis_name="s")

    @functools.partial(
        pl.kernel,
        mesh=mesh,
        out_type=jax.ShapeDtypeStruct((n_edges, d), jnp.float32),
        scratch_types=[
            pltpu.VMEM((chunk,), jnp.int32),
            pltpu.VMEM((chunk,), jnp.int32),
            pltpu.VMEM((chunk,), jnp.int32),
            pltpu.VMEM((chunk,), jnp.int32),
            pltpu.VMEM((chunk, d), jnp.float32),
            pltpu.VMEM((chunk, d), jnp.float32),
            pltpu.VMEM((chunk, d), jnp.float32),
            pltpu.VMEM((chunk, d), jnp.float32),
        ] + [pltpu.SemaphoreType.DMA] * 10,
    )
    def gather_add(p_hbm, q_hbm, src_hbm, tgt_hbm, out_hbm,
                   is0, is1, it0, it1, rp0, rp1, rq0, rq1,
                   sis0, sis1, sit0, sit1, sgp0, sgp1, sgq0, sgq1, sw0, sw1):
        idx_s = [is0, is1]
        idx_t = [it0, it1]
        rows_p = [rp0, rp1]
        rows_q = [rq0, rq1]
        sem_is = [sis0, sis1]
        sem_it = [sit0, sit1]
        sem_gp = [sgp0, sgp1]
        sem_gq = [sgq0, sgq1]
        sem_w = [sw0, sw1]

        wid = lax.axis_index("s") * nc + lax.axis_index("c")
        base = wid * per_w

        def fire_idx(c, b):
            eb = base + c * chunk
            pltpu.async_copy(src_hbm.at[pl.ds(eb, chunk)], idx_s[b], sem_is[b])
            pltpu.async_copy(tgt_hbm.at[pl.ds(eb, chunk)], idx_t[b], sem_it[b])

        def wait_idx(c, b):
            eb = base + c * chunk
            pltpu.make_async_copy(
                src_hbm.at[pl.ds(eb, chunk)], idx_s[b], sem_is[b]).wait()
            pltpu.make_async_copy(
                tgt_hbm.at[pl.ds(eb, chunk)], idx_t[b], sem_it[b]).wait()

        def fire_gather(b):
            pltpu.async_copy(p_hbm.at[idx_s[b]], rows_p[b], sem_gp[b])
            pltpu.async_copy(q_hbm.at[idx_t[b]], rows_q[b], sem_gq[b])

        def wait_gather(b):
            pltpu.make_async_copy(p_hbm.at[idx_s[b]], rows_p[b], sem_gp[b]).wait()
            pltpu.make_async_copy(q_hbm.at[idx_t[b]], rows_q[b], sem_gq[b]).wait()

        def fire_wb(c, b):
            eb = base + c * chunk
            pltpu.async_copy(rows_p[b], out_hbm.at[pl.ds(eb, chunk)], sem_w[b])

        def wait_wb(c, b):
            eb = base + c * chunk
            pltpu.make_async_copy(
                rows_p[b], out_hbm.at[pl.ds(eb, chunk)], sem_w[b]).wait()

        fire_idx(0, 0)
        fire_idx(1, 1)
        wait_idx(0, 0)
        fire_gather(0)

        def pair(hh, carry):
            for b in range(2):
                h = 2 * hh + b
                nb = 1 - b

                @pl.when(h < n_chunks)
                def _():
                    wait_gather(b)

                    @pl.when(h + 2 < n_chunks)
                    def _():
                        fire_idx(h + 2, b)

                    @pl.when(h >= 1)
                    def _():
                        wait_wb(h - 1, nb)

                    @pl.when(h + 1 < n_chunks)
                    def _():
                        wait_idx(h + 1, nb)
                        fire_gather(nb)

                    def add_row(i, c2):
                        for j in range(d // 16):
                            sl = pl.ds(j * 16, 16)
                            rows_p[b][i, sl] = rows_p[b][i, sl] + rows_q[b][i, sl]
                        return c2

                    lax.fori_loop(0, chunk, add_row, 0)
                    fire_wb(h, b)
            return carry

        lax.fori_loop(0, n_pairs, pair, 0)
        wait_wb(n_chunks - 1, (n_chunks - 1) % 2)

    return gather_add


def _msg_body(e_ref, g_ref, we_ref, o_ref):
    dn = (((1,), (1,)), ((), ()))
    o_ref[...] = g_ref[...] + lax.dot_general(
        e_ref[...], we_ref[...], dn, preferred_element_type=jnp.float32)


def _msg_body_aliased(e_ref, g_ref, we_ref, prev_ref, o_ref):
    del prev_ref
    _msg_body(e_ref, g_ref, we_ref, o_ref)


def _block_off_map(block_off):
    return lambda i: (i + block_off, 0)


def _compute_msg_half(edge_full, g_half, we, block, block_off, prev=None):
    e_total, d = edge_full.shape
    grid = (g_half.shape[0] // block,)
    in_specs = [
        pl.BlockSpec((block, d), _block_off_map(block_off)),
        pl.BlockSpec((block, d), lambda i: (i, 0)),
        pl.BlockSpec((d, d), lambda i: (0, 0)),
    ]
    args = [edge_full, g_half, we]
    body = _msg_body
    kwargs = {}
    if prev is not None:
        in_specs.append(pl.BlockSpec(memory_space=pl.ANY))
        args.append(prev)
        body = _msg_body_aliased
        kwargs["input_output_aliases"] = {3: 0}
    return pl.pallas_call(
        body,
        grid=grid,
        in_specs=in_specs,
        out_specs=pl.BlockSpec((block, d), _block_off_map(block_off)),
        out_shape=jax.ShapeDtypeStruct((e_total, d), jnp.float32),
        **kwargs,
    )(*args)


def kernel(x, edge_index, edge_attr, W):
    n, d = x.shape
    e = edge_attr.shape[0]
    ws = W[:, :d]
    wt = W[:, d:2 * d]
    we = W[:, 2 * d:]

    p, q = _compute_pq(x, ws, wt)
    src = edge_index[0]
    tgt = edge_index[1]

    block = 3200
    n_seg = 5
    seg = e // n_seg
    gather_seg = _make_gather_add(seg, d, chunk=80)

    gs = [gather_seg(p, q, src[k * seg:(k + 1) * seg],
                     tgt[k * seg:(k + 1) * seg]) for k in range(n_seg)]
    out = _compute_msg_half(edge_attr, gs[0], we, block, 0)
    for k in range(1, n_seg):
        out = _compute_msg_half(edge_attr, gs[k], we, block,
                                k * seg // block, prev=out)
    return out


# speedup vs baseline: 1.2183x; 1.0085x over previous
import functools

import jax
import jax.numpy as jnp
from jax import lax
from jax.experimental import pallas as pl
from jax.experimental.pallas import tpu as pltpu
from jax.experimental.pallas import tpu_sc as plsc


def _pq_body(x_ref, ws_ref, wt_ref, p_ref, q_ref):
    xv = x_ref[...]
    dn = (((1,), (1,)), ((), ()))
    p_ref[...] = lax.dot_general(xv, ws_ref[...], dn,
                                 preferred_element_type=jnp.float32)
    q_ref[...] = lax.dot_general(xv, wt_ref[...], dn,
                                 preferred_element_type=jnp.float32)


def _compute_pq(x, ws, wt):
    n, d = x.shape
    return pl.pallas_call(
        _pq_body,
        out_shape=(
            jax.ShapeDtypeStruct((n, d), jnp.float32),
            jax.ShapeDtypeStruct((n, d), jnp.float32),
        ),
    )(x, ws, wt)


def _make_gather_add(n_edges, d, chunk):
    info = plsc.get_sparse_core_info()
    nc, ns = info.num_cores, info.num_subcores
    nw = nc * ns
    per_w = n_edges // nw
    n_chunks = per_w // chunk
    n_pairs = (n_chunks + 2) // 2
    assert per_w % chunk == 0 and n_edges % nw == 0
    assert chunk % 8 == 0 and chunk <= 128

    mesh = plsc.VectorSubcoreMesh(core_axis_name="c", subcore_axis_name="s")

    @functools.partial(
        pl.kernel,
        mesh=mesh,
        out_type=jax.ShapeDtypeStruct((n_edges, d), jnp.float32),
        scratch_types=[
            pltpu.VMEM((chunk,), jnp.int32),
            pltpu.VMEM((chunk,), jnp.int32),
            pltpu.VMEM((chunk,), jnp.int32),
            pltpu.VMEM((chunk,), jnp.int32),
            pltpu.VMEM((chunk, d), jnp.float32),
            pltpu.VMEM((chunk, d), jnp.float32),
            pltpu.VMEM((chunk, d), jnp.float32),
            pltpu.VMEM((chunk, d), jnp.float32),
        ] + [pltpu.SemaphoreType.DMA] * 10,
    )
    def gather_add(p_hbm, q_hbm, src_hbm, tgt_hbm, out_hbm,
                   is0, is1, it0, it1, rp0, rp1, rq0, rq1,
                   sis0, sis1, sit0, sit1, sgp0, sgp1, sgq0, sgq1, sw0, sw1):
        idx_s = [is0, is1]
        idx_t = [it0, it1]
        rows_p = [rp0, rp1]
        rows_q = [rq0, rq1]
        sem_is = [sis0, sis1]
        sem_it = [sit0, sit1]
        sem_gp = [sgp0, sgp1]
        sem_gq = [sgq0, sgq1]
        sem_w = [sw0, sw1]

        wid = lax.axis_index("s") * nc + lax.axis_index("c")
        base = wid * per_w

        def fire_idx(c, b):
            eb = base + c * chunk
            pltpu.async_copy(src_hbm.at[pl.ds(eb, chunk)], idx_s[b], sem_is[b])
            pltpu.async_copy(tgt_hbm.at[pl.ds(eb, chunk)], idx_t[b], sem_it[b])

        def wait_idx(c, b):
            eb = base + c * chunk
            pltpu.make_async_copy(
                src_hbm.at[pl.ds(eb, chunk)], idx_s[b], sem_is[b]).wait()
            pltpu.make_async_copy(
                tgt_hbm.at[pl.ds(eb, chunk)], idx_t[b], sem_it[b]).wait()

        def fire_gather(b):
            pltpu.async_copy(p_hbm.at[idx_s[b]], rows_p[b], sem_gp[b])
            pltpu.async_copy(q_hbm.at[idx_t[b]], rows_q[b], sem_gq[b])

        def wait_gather(b):
            pltpu.make_async_copy(p_hbm.at[idx_s[b]], rows_p[b], sem_gp[b]).wait()
            pltpu.make_async_copy(q_hbm.at[idx_t[b]], rows_q[b], sem_gq[b]).wait()

        def fire_wb(c, b):
            eb = base + c * chunk
            pltpu.async_copy(rows_p[b], out_hbm.at[pl.ds(eb, chunk)], sem_w[b])

        def wait_wb(c, b):
            eb = base + c * chunk
            pltpu.make_async_copy(
                rows_p[b], out_hbm.at[pl.ds(eb, chunk)], sem_w[b]).wait()

        fire_idx(0, 0)
        fire_idx(1, 1)
        wait_idx(0, 0)
        fire_gather(0)

        def pair(hh, carry):
            for b in range(2):
                h = 2 * hh + b
                nb = 1 - b

                @pl.when(h < n_chunks)
                def _():
                    wait_gather(b)

                    @pl.when(h + 2 < n_chunks)
                    def _():
                        fire_idx(h + 2, b)

                    @pl.when(h >= 1)
                    def _():
                        wait_wb(h - 1, nb)

                    @pl.when(h + 1 < n_chunks)
                    def _():
                        wait_idx(h + 1, nb)
                        fire_gather(nb)

                    def add_row(i, c2):
                        for j in range(d // 16):
                            sl = pl.ds(j * 16, 16)
                            rows_p[b][i, sl] = rows_p[b][i, sl] + rows_q[b][i, sl]
                        return c2

                    lax.fori_loop(0, chunk, add_row, 0)
                    fire_wb(h, b)
            return carry

        lax.fori_loop(0, n_pairs, pair, 0)
        wait_wb(n_chunks - 1, (n_chunks - 1) % 2)

    return gather_add


def _msg_body(e_ref, g_ref, we_ref, o_ref):
    dn = (((1,), (1,)), ((), ()))
    o_ref[...] = g_ref[...] + lax.dot_general(
        e_ref[...], we_ref[...], dn, preferred_element_type=jnp.float32)


def _msg_body_aliased(e_ref, g_ref, we_ref, prev_ref, o_ref):
    del prev_ref
    _msg_body(e_ref, g_ref, we_ref, o_ref)


def _block_off_map(block_off):
    return lambda i: (i + block_off, 0)


def _compute_msg_half(edge_full, g_half, we, block, block_off, prev=None):
    e_total, d = edge_full.shape
    grid = (g_half.shape[0] // block,)
    in_specs = [
        pl.BlockSpec((block, d), _block_off_map(block_off)),
        pl.BlockSpec((block, d), lambda i: (i, 0)),
        pl.BlockSpec((d, d), lambda i: (0, 0)),
    ]
    args = [edge_full, g_half, we]
    body = _msg_body
    kwargs = {}
    if prev is not None:
        in_specs.append(pl.BlockSpec(memory_space=pl.ANY))
        args.append(prev)
        body = _msg_body_aliased
        kwargs["input_output_aliases"] = {3: 0}
    return pl.pallas_call(
        body,
        grid=grid,
        in_specs=in_specs,
        out_specs=pl.BlockSpec((block, d), _block_off_map(block_off)),
        out_shape=jax.ShapeDtypeStruct((e_total, d), jnp.float32),
        **kwargs,
    )(*args)


def kernel(x, edge_index, edge_attr, W):
    n, d = x.shape
    e = edge_attr.shape[0]
    ws = W[:, :d]
    wt = W[:, d:2 * d]
    we = W[:, 2 * d:]

    p, q = _compute_pq(x, ws, wt)
    src = edge_index[0]
    tgt = edge_index[1]

    block = 6400
    unit = 12800
    segs = [2 * unit, 5 * unit, 6 * unit, 6 * unit, 6 * unit]
    assert sum(segs) == e
    gather_for = {s: _make_gather_add(s, d, chunk=80) for s in set(segs)}

    offs = [sum(segs[:k]) for k in range(len(segs))]
    gs = [gather_for[s](p, q, src[o:o + s], tgt[o:o + s])
          for o, s in zip(offs, segs)]
    out = _compute_msg_half(edge_attr, gs[0], we, block, 0)
    for k in range(1, len(segs)):
        out = _compute_msg_half(edge_attr, gs[k], we, block,
                                offs[k] // block, prev=out)
    return out
